# baseline (speedup 1.0000x reference)
import functools

import jax
import jax.numpy as jnp
from jax import lax
from jax.experimental import pallas as pl
from jax.experimental.pallas import tpu as pltpu
from jax.experimental.pallas import tpu_sc as plsc

B = 16384
D = 64
NC = 2
NS = 16
NW = NC * NS
BPW = B // NW
CH = 128
NCH = BPW // CH

TC_ROWS = 8192


def _sc_body(tab_hbm, idx_hbm, out_hbm, idx_v, rows_v, sem_idx, sem_gat,
             sem_out):
    wid = lax.axis_index("s") * NC + lax.axis_index("c")
    base = wid * BPW

    pltpu.async_copy(idx_hbm.at[wid], idx_v, sem_idx).wait()

    gathers = []
    for j in range(NCH):
        gathers.append(pltpu.async_copy(
            tab_hbm.at[idx_v.at[j]], rows_v.at[pl.ds(j * CH, CH)], sem_gat))
    for c in gathers:
        c.wait()

    pltpu.async_copy(rows_v, out_hbm.at[pl.ds(base, BPW)], sem_out).wait()


V = 100000
TR_COLS = 32768


def _tr_body(inT_ref, out_ref):
    out_ref[:, :D] = inT_ref[...].T


def _tc_body(gu_ref, gi_ref, xui_ref, guT_ref, giT_ref):
    gu = gu_ref[:, :D]
    gi = gi_ref[:, :D]
    xui_ref[...] = jnp.sum(gu * gi, axis=1)
    guT_ref[...] = gu.T
    giT_ref[...] = gi.T


@jax.jit
def _run(Gu, Gi, user_r, item_r):
    mesh = plsc.VectorSubcoreMesh(core_axis_name="c", subcore_axis_name="s")

    tr_fn = pl.pallas_call(
        _tr_body,
        grid=((V + TR_COLS - 1) // TR_COLS,),
        in_specs=[pl.BlockSpec((D, TR_COLS), lambda i: (0, i))],
        out_specs=pl.BlockSpec((TR_COLS, 128), lambda i: (i, 0)),
        out_shape=jax.ShapeDtypeStruct((V, 128), jnp.float32),
    )
    Gu_lin = tr_fn(Gu.T)
    Gi_lin = tr_fn(Gi.T)

    gather_fn = pl.kernel(
        _sc_body,
        out_type=[jax.ShapeDtypeStruct((B, 128), jnp.float32)],
        mesh=mesh,
        compiler_params=pltpu.CompilerParams(use_tc_tiling_on_sc=False),
        scratch_types=[
            pltpu.VMEM((NCH, CH), jnp.int32),
            pltpu.VMEM((BPW, 128), jnp.float32),
            pltpu.SemaphoreType.DMA,
            pltpu.SemaphoreType.DMA,
            pltpu.SemaphoreType.DMA,
        ],
    )
    (gamma_u,) = gather_fn(Gu_lin, user_r)
    (gamma_i,) = gather_fn(Gi_lin, item_r)

    xui, guT, giT = pl.pallas_call(
        _tc_body,
        grid=(B // TC_ROWS,),
        in_specs=[
            pl.BlockSpec((TC_ROWS, 128), lambda i: (i, 0)),
            pl.BlockSpec((TC_ROWS, 128), lambda i: (i, 0)),
        ],
        out_specs=[
            pl.BlockSpec((TC_ROWS,), lambda i: (i,)),
            pl.BlockSpec((D, TC_ROWS), lambda i: (0, i)),
            pl.BlockSpec((D, TC_ROWS), lambda i: (0, i)),
        ],
        out_shape=[
            jax.ShapeDtypeStruct((B,), jnp.float32),
            jax.ShapeDtypeStruct((D, B), jnp.float32),
            jax.ShapeDtypeStruct((D, B), jnp.float32),
        ],
    )(gamma_u, gamma_i)

    return xui, guT.T, giT.T


def kernel(Gu, Gi, user, item):
    user_r = user.astype(jnp.int32).reshape(NW, NCH, CH)
    item_r = item.astype(jnp.int32).reshape(NW, NCH, CH)
    xui, gamma_u, gamma_i = _run(Gu, Gi, user_r, item_r)
    return (xui, gamma_u, gamma_i)


# speedup vs baseline: 1.0078x; 1.0078x over previous
import functools

import jax
import jax.numpy as jnp
from jax import lax
from jax.experimental import pallas as pl
from jax.experimental.pallas import tpu as pltpu
from jax.experimental.pallas import tpu_sc as plsc

B = 16384
D = 64
NC = 2
NS = 16
NW = NC * NS
BPW = B // NW
CH = 128
NCH = BPW // CH

TC_ROWS = 4096


def _sc_body(tab_hbm, idx_hbm, out_hbm, idx_v, rows_v, sem_idx, sem_gat,
             sem_out):
    wid = lax.axis_index("s") * NC + lax.axis_index("c")
    base = wid * BPW

    pltpu.async_copy(idx_hbm.at[wid], idx_v, sem_idx).wait()

    gathers = []
    for j in range(NCH):
        gathers.append(pltpu.async_copy(
            tab_hbm.at[idx_v.at[j]], rows_v.at[pl.ds(j * CH, CH)], sem_gat))
    for c in gathers:
        c.wait()

    pltpu.async_copy(rows_v, out_hbm.at[pl.ds(base, BPW)], sem_out).wait()


V = 100000
TR_COLS = 16384


def _tr_body(inT_ref, out_ref):
    out_ref[:, :D] = inT_ref[...].T


def _tc_body(gu_ref, gi_ref, xui_ref, guT_ref, giT_ref):
    gu = gu_ref[:, :D]
    gi = gi_ref[:, :D]
    xui_ref[...] = jnp.sum(gu * gi, axis=1)
    guT_ref[...] = gu.T
    giT_ref[...] = gi.T


@jax.jit
def _run(Gu, Gi, user_r, item_r):
    mesh = plsc.VectorSubcoreMesh(core_axis_name="c", subcore_axis_name="s")

    tr_fn = pl.pallas_call(
        _tr_body,
        grid=((V + TR_COLS - 1) // TR_COLS,),
        in_specs=[pl.BlockSpec((D, TR_COLS), lambda i: (0, i))],
        out_specs=pl.BlockSpec((TR_COLS, 128), lambda i: (i, 0)),
        out_shape=jax.ShapeDtypeStruct((V, 128), jnp.float32),
    )
    Gu_lin = tr_fn(Gu.T)
    Gi_lin = tr_fn(Gi.T)

    gather_fn = pl.kernel(
        _sc_body,
        out_type=[jax.ShapeDtypeStruct((B, 128), jnp.float32)],
        mesh=mesh,
        compiler_params=pltpu.CompilerParams(use_tc_tiling_on_sc=False),
        scratch_types=[
            pltpu.VMEM((NCH, CH), jnp.int32),
            pltpu.VMEM((BPW, 128), jnp.float32),
            pltpu.SemaphoreType.DMA,
            pltpu.SemaphoreType.DMA,
            pltpu.SemaphoreType.DMA,
        ],
    )
    (gamma_u,) = gather_fn(Gu_lin, user_r)
    (gamma_i,) = gather_fn(Gi_lin, item_r)

    xui, guT, giT = pl.pallas_call(
        _tc_body,
        grid=(B // TC_ROWS,),
        in_specs=[
            pl.BlockSpec((TC_ROWS, 128), lambda i: (i, 0)),
            pl.BlockSpec((TC_ROWS, 128), lambda i: (i, 0)),
        ],
        out_specs=[
            pl.BlockSpec((TC_ROWS,), lambda i: (i,)),
            pl.BlockSpec((D, TC_ROWS), lambda i: (0, i)),
            pl.BlockSpec((D, TC_ROWS), lambda i: (0, i)),
        ],
        out_shape=[
            jax.ShapeDtypeStruct((B,), jnp.float32),
            jax.ShapeDtypeStruct((D, B), jnp.float32),
            jax.ShapeDtypeStruct((D, B), jnp.float32),
        ],
    )(gamma_u, gamma_i)

    return xui, guT.T, giT.T


def kernel(Gu, Gi, user, item):
    user_r = user.astype(jnp.int32).reshape(NW, NCH, CH)
    item_r = item.astype(jnp.int32).reshape(NW, NCH, CH)
    xui, gamma_u, gamma_i = _run(Gu, Gi, user_r, item_r)
    return (xui, gamma_u, gamma_i)
